# Initial kernel scaffold; baseline (speedup 1.0000x reference)
import jax
import jax.numpy as jnp
from jax.experimental import pallas as pl


def kernel(x, edge_index, edge_attr, batch, W_enc, b_enc, W_self, W_nei, W_edge, W1, b1, gamma, beta, W2, b2):
    raise NotImplementedError("write your pallas kernel here")


# speedup vs baseline: 1.6405x; 1.6405x over previous
import functools

import jax
import jax.numpy as jnp
from jax import lax
from jax.experimental import pallas as pl
from jax.experimental.pallas import tpu as pltpu
from jax.experimental.pallas import tpu_sc as plsc

_NC = 2
_NS = 16
_NW = _NC * _NS
_K = 64
_GRP = 8


def _sc_aggregate(x, src_b, dst_b, ea_b, npad):
    d = x.shape[1]
    da = ea_b.shape[-1]
    chunks = ea_b.shape[0] // _NW
    rows_per_tile = npad // _NS
    nslab = rows_per_tile // _K
    mesh = plsc.VectorSubcoreMesh(core_axis_name="c", subcore_axis_name="s")

    @functools.partial(
        pl.kernel,
        out_type=(
            jax.ShapeDtypeStruct((_NC, npad, d), jnp.float32),
            jax.ShapeDtypeStruct((_NC, npad, da), jnp.float32),
        ),
        mesh=mesh,
        compiler_params=pltpu.CompilerParams(use_tc_tiling_on_sc=False),
        scratch_types=[
            pltpu.VMEM((_K,), jnp.int32),
            pltpu.VMEM((_K,), jnp.int32),
            pltpu.VMEM((_K, d), jnp.float32),
            pltpu.VMEM((_K, da), jnp.float32),
            pltpu.VMEM_SHARED((npad, d), jnp.float32),
            pltpu.VMEM_SHARED((npad, da), jnp.float32),
            pltpu.SemaphoreType.DMA,
        ],
    )
    def sc(x_hbm, src_hbm, dst_hbm, ea_hbm,
           outx_hbm, oute_hbm,
           sidx_v, didx_v, rows_v, ea_v,
           accx_sh, acce_sh, sem):
        cid = lax.axis_index("c")
        sid = lax.axis_index("s")
        wid = sid * _NC + cid
        r0 = sid * rows_per_tile
        zero16 = jnp.zeros((16,), jnp.float32)

        def zrow(i, c):
            for c8 in range(d // 16):
                rows_v[i, pl.ds(c8 * 16, 16)] = zero16
            for c8 in range(da // 16):
                ea_v[i, pl.ds(c8 * 16, 16)] = zero16
            return c
        lax.fori_loop(0, _K, zrow, 0)

        def zslab(i, c):
            base = r0 + i * _K
            pltpu.sync_copy(rows_v, accx_sh.at[pl.ds(base, _K)])
            pltpu.sync_copy(ea_v, acce_sh.at[pl.ds(base, _K)])
            return c
        lax.fori_loop(0, nslab, zslab, 0)

        plsc.subcore_barrier()

        def chunk(c, carry):
            base = (wid * chunks + c) * _K
            pltpu.sync_copy(src_hbm.at[pl.ds(base, _K)], sidx_v)
            pltpu.sync_copy(dst_hbm.at[pl.ds(base, _K)], didx_v)
            pltpu.async_copy(x_hbm.at[sidx_v], rows_v, sem).wait()
            pltpu.sync_copy(rows_v, accx_sh.at[didx_v], add=True)
            pltpu.sync_copy(ea_hbm.at[wid * chunks + c], ea_v)
            pltpu.sync_copy(ea_v, acce_sh.at[didx_v], add=True)
            return carry

        lax.fori_loop(0, chunks, chunk, 0)
        plsc.subcore_barrier()

        def rslab(i, c):
            base = r0 + i * _K
            pltpu.sync_copy(accx_sh.at[pl.ds(base, _K)], rows_v)
            pltpu.sync_copy(rows_v, outx_hbm.at[cid, pl.ds(base, _K)])
            pltpu.sync_copy(acce_sh.at[pl.ds(base, _K)], ea_v)
            pltpu.sync_copy(ea_v, oute_hbm.at[cid, pl.ds(base, _K)])
            return c
        lax.fori_loop(0, nslab, rslab, 0)

    return sc(x, src_b, dst_b, ea_b)


def _dense(n, rb, de, x, agx, age, W_enc, b_enc, W_self, W_nei, W_edge,
           W1, b1, gamma, beta, W2, b2):
    d = x.shape[1]
    da = age.shape[2]
    h_w = W1.shape[1]
    out_w = W2.shape[1]
    nb = n // rb
    hp = lax.Precision.HIGHEST

    def body1(x_ref, ax_ref, ae_ref, wenc_ref, benc_ref, wself_ref,
              wnei_ref, wedge_ref, w1_ref, b1_ref, t_ref, s_ref):
        i = pl.program_id(0)
        h = (jnp.dot(x_ref[...], wenc_ref[...], precision=hp)
             + benc_ref[...][None, :])
        ax = ax_ref[0] + ax_ref[1]
        ae_all = ae_ref[0] + ae_ref[1]
        ae = ae_all[:, :de]
        deg_v = ae_all[:, de]
        wxn = jnp.dot(wenc_ref[...], wnei_ref[...], precision=hp)
        bnei = jnp.dot(benc_ref[...].reshape(1, -1), wnei_ref[...],
                       precision=hp)
        agg = (jnp.dot(ax, wxn, precision=hp)
               + deg_v.reshape(-1, 1) * bnei
               + jnp.dot(ae, wedge_ref[...], precision=hp))
        h2 = jnp.maximum(jnp.dot(h, wself_ref[...], precision=hp) + agg, 0.0)
        t = jnp.dot(h2, w1_ref[...], precision=hp) + b1_ref[...][None, :]
        t_ref[...] = t

        @pl.when(i == 0)
        def _():
            s_ref[...] = jnp.zeros_like(s_ref)

        s_ref[0, :] += jnp.sum(t, axis=0)
        s_ref[1, :] += jnp.sum(t * t, axis=0)

    t_full, sums = pl.pallas_call(
        body1,
        grid=(nb,),
        in_specs=[
            pl.BlockSpec((rb, d), lambda i: (i, 0)),
            pl.BlockSpec((2, rb, d), lambda i: (0, i, 0)),
            pl.BlockSpec((2, rb, da), lambda i: (0, i, 0)),
            pl.BlockSpec((d, d), lambda i: (0, 0)),
            pl.BlockSpec((d,), lambda i: (0,)),
            pl.BlockSpec((d, d), lambda i: (0, 0)),
            pl.BlockSpec((d, d), lambda i: (0, 0)),
            pl.BlockSpec((de, d), lambda i: (0, 0)),
            pl.BlockSpec((d, h_w), lambda i: (0, 0)),
            pl.BlockSpec((h_w,), lambda i: (0,)),
        ],
        out_specs=[
            pl.BlockSpec((rb, h_w), lambda i: (i, 0)),
            pl.BlockSpec((2, h_w), lambda i: (0, 0)),
        ],
        out_shape=[
            jax.ShapeDtypeStruct((n, h_w), jnp.float32),
            jax.ShapeDtypeStruct((2, h_w), jnp.float32),
        ],
    )(x, agx, age, W_enc, b_enc, W_self, W_nei, W_edge, W1, b1)

    def body2(t_ref, s_ref, g_ref, bt_ref, w2_ref, b2_ref, o_ref):
        t = t_ref[...]
        mean = s_ref[0, :][None, :] * (1.0 / n)
        var = s_ref[1, :][None, :] * (1.0 / n) - mean * mean
        tn = ((t - mean) * lax.rsqrt(var + 1e-5) * g_ref[...][None, :]
              + bt_ref[...][None, :])
        h3 = jnp.maximum(tn, 0.0)
        o_ref[...] = (jnp.dot(h3, w2_ref[...], precision=hp)
                      + b2_ref[...][None, :])

    return pl.pallas_call(
        body2,
        grid=(nb,),
        in_specs=[
            pl.BlockSpec((rb, h_w), lambda i: (i, 0)),
            pl.BlockSpec((2, h_w), lambda i: (0, 0)),
            pl.BlockSpec((h_w,), lambda i: (0,)),
            pl.BlockSpec((h_w,), lambda i: (0,)),
            pl.BlockSpec((h_w, out_w), lambda i: (0, 0)),
            pl.BlockSpec((out_w,), lambda i: (0,)),
        ],
        out_specs=pl.BlockSpec((rb, out_w), lambda i: (i, 0)),
        out_shape=jax.ShapeDtypeStruct((n, out_w), jnp.float32),
    )(t_full, sums, gamma, beta, W2, b2)


def kernel(x, edge_index, edge_attr, batch, W_enc, b_enc, W_self, W_nei,
           W_edge, W1, b1, gamma, beta, W2, b2):
    n, d = x.shape
    e, de = edge_attr.shape
    blk = _NS * _K
    npad = -(-n // blk) * blk
    if npad == n:
        npad += blk
    chunks = -(-e // (_NW * _K))
    chunks = -(-chunks // _GRP) * _GRP
    epad = _NW * chunks * _K

    src = edge_index[0].astype(jnp.int32)
    dst = edge_index[1].astype(jnp.int32)
    pad = epad - e
    src_b = jnp.concatenate([src, jnp.zeros((pad,), jnp.int32)])
    dst_b = jnp.concatenate([dst, jnp.full((pad,), npad - 1, jnp.int32)])
    da = -(-(de + 1) // 16) * 16
    ea = edge_attr.astype(jnp.float32)
    ea = jnp.concatenate(
        [ea, jnp.ones((e, 1), jnp.float32),
         jnp.zeros((e, da - de - 1), jnp.float32)], axis=1)
    ea_b = jnp.concatenate([ea, jnp.zeros((pad, da), jnp.float32)])
    ea_b = ea_b.reshape(_NW * chunks, _K, da)

    agx, age = _sc_aggregate(x, src_b, dst_b, ea_b, npad)
    agx = agx[:, :n, :]
    age = age[:, :n, :]

    rb = n
    for cand in range(min(1024, n), 0, -1):
        if n % cand == 0:
            rb = cand
            break
    out = _dense(n, rb, de, x, agx, age, W_enc, b_enc, W_self, W_nei,
                 W_edge, W1, b1, gamma, beta, W2, b2)
    return out.reshape(-1, 8, 4)


# speedup vs baseline: 2.5250x; 1.5392x over previous
import functools

import jax
import jax.numpy as jnp
from jax import lax
from jax.experimental import pallas as pl
from jax.experimental.pallas import tpu as pltpu
from jax.experimental.pallas import tpu_sc as plsc

_NC = 2
_NS = 16
_K = 128
_GRP = 8


def _sc_aggregate(x2, src2, dst2, ea2, npad):
    dh = x2.shape[1]
    dep = ea2.shape[-1]
    chunks = src2.shape[0] // (_NC * _NS)
    rows_per_tile = npad // _NS
    nslab = rows_per_tile // _K
    mesh = plsc.VectorSubcoreMesh(core_axis_name="c", subcore_axis_name="s")

    @functools.partial(
        pl.kernel,
        out_type=(
            jax.ShapeDtypeStruct((_NC, npad, dh), jnp.float32),
            jax.ShapeDtypeStruct((_NC, npad, dep), jnp.float32),
        ),
        mesh=mesh,
        compiler_params=pltpu.CompilerParams(use_tc_tiling_on_sc=False),
        scratch_types=[
            pltpu.VMEM((_GRP, _K), jnp.int32),
            pltpu.VMEM((_GRP, _K), jnp.int32),
            pltpu.VMEM((_K, dh), jnp.float32),
            pltpu.VMEM((_K, dh), jnp.float32),
            pltpu.VMEM((_K, dep), jnp.float32),
            pltpu.VMEM((_K, dep), jnp.float32),
            pltpu.VMEM_SHARED((npad, dh), jnp.float32),
            pltpu.VMEM_SHARED((npad, dep), jnp.float32),
            pltpu.SemaphoreType.DMA,
            pltpu.SemaphoreType.DMA,
            pltpu.SemaphoreType.DMA,
            pltpu.SemaphoreType.DMA,
            pltpu.SemaphoreType.DMA,
            pltpu.SemaphoreType.DMA,
            pltpu.SemaphoreType.DMA,
            pltpu.SemaphoreType.DMA,
        ],
    )
    def sc(x_hbm, src_hbm, dst_hbm, ea_hbm,
           outx_hbm, oute_hbm,
           sidx_g, didx_g, rows0_v, rows1_v, ea0_v, ea1_v,
           accx_sh, acce_sh,
           gsem0, gsem1, esem0, esem1, sxsem0, sxsem1, sesem0, sesem1):
        cid = lax.axis_index("c")
        sid = lax.axis_index("s")
        row0 = (cid * _NS + sid) * chunks
        r0 = sid * rows_per_tile
        zero16 = jnp.zeros((16,), jnp.float32)
        rows = [rows0_v, rows1_v]
        eab = [ea0_v, ea1_v]
        gsem = [gsem0, gsem1]
        esem = [esem0, esem1]
        sxsem = [sxsem0, sxsem1]
        sesem = [sesem0, sesem1]

        def zrow(i, c):
            for c8 in range(dh // 16):
                rows0_v[i, pl.ds(c8 * 16, 16)] = zero16
            for c8 in range(dep // 16):
                ea0_v[i, pl.ds(c8 * 16, 16)] = zero16
            return c
        lax.fori_loop(0, _K, zrow, 0)

        def zslab(i, c):
            base = r0 + i * _K
            pltpu.sync_copy(rows0_v, accx_sh.at[pl.ds(base, _K)])
            pltpu.sync_copy(ea0_v, acce_sh.at[pl.ds(base, _K)])
            return c
        lax.fori_loop(0, nslab, zslab, 0)

        plsc.subcore_barrier()

        def group(g, carry):
            gb = row0 + g * _GRP
            pltpu.sync_copy(src_hbm.at[pl.ds(gb, _GRP)], sidx_g)
            pltpu.sync_copy(dst_hbm.at[pl.ds(gb, _GRP)], didx_g)

            gds = {}
            eds = {}
            pend_sx = {}
            pend_se = {}
            gds[0] = pltpu.async_copy(x_hbm.at[sidx_g.at[0]], rows[0],
                                      gsem[0])
            eds[0] = pltpu.async_copy(ea_hbm.at[gb], eab[0], esem[0])
            for j in range(_GRP):
                b = j & 1
                if j + 1 < _GRP:
                    b2 = (j + 1) & 1
                    if j >= 1:
                        pend_sx[j - 1].wait()
                        pend_se[j - 1].wait()
                    gds[j + 1] = pltpu.async_copy(
                        x_hbm.at[sidx_g.at[j + 1]], rows[b2], gsem[b2])
                    eds[j + 1] = pltpu.async_copy(
                        ea_hbm.at[gb + j + 1], eab[b2], esem[b2])
                gds[j].wait()
                pend_sx[j] = pltpu.async_copy(
                    rows[b], accx_sh.at[didx_g.at[j]], sxsem[b], add=True)
                eds[j].wait()
                pend_se[j] = pltpu.async_copy(
                    eab[b], acce_sh.at[didx_g.at[j]], sesem[b], add=True)
            pend_sx[_GRP - 2].wait()
            pend_se[_GRP - 2].wait()
            pend_sx[_GRP - 1].wait()
            pend_se[_GRP - 1].wait()
            return carry

        lax.fori_loop(0, chunks // _GRP, group, 0)
        plsc.subcore_barrier()

        def rslab(i, c):
            base = r0 + i * _K
            pltpu.sync_copy(accx_sh.at[pl.ds(base, _K)], rows0_v)
            pltpu.sync_copy(rows0_v, outx_hbm.at[cid, pl.ds(base, _K)])
            pltpu.sync_copy(acce_sh.at[pl.ds(base, _K)], ea0_v)
            pltpu.sync_copy(ea0_v, oute_hbm.at[cid, pl.ds(base, _K)])
            return c
        lax.fori_loop(0, nslab, rslab, 0)

    return sc(x2, src2, dst2, ea2)


def _dense(n, rb, de, x, agx, age, W_enc, b_enc, W_self, W_nei, W_edge,
           W1, b1, gamma, beta, W2, b2):
    d = x.shape[1]
    dh = agx.shape[2]
    dep = age.shape[2]
    h_w = W1.shape[1]
    out_w = W2.shape[1]
    nb = n // rb
    hp = lax.Precision.HIGHEST

    def body1(x_ref, ax_ref, ae_ref, wenc_ref, benc_ref, wself_ref,
              wnei_ref, wedge_ref, w1_ref, b1_ref, t_ref, s_ref):
        i = pl.program_id(0)
        h = (jnp.dot(x_ref[...], wenc_ref[...], precision=hp)
             + benc_ref[...][None, :])
        ae = ae_ref[0][:, :de]
        deg_v = ae_ref[1][:, 0]
        wxn = jnp.dot(wenc_ref[...], wnei_ref[...], precision=hp)
        bnei = jnp.dot(benc_ref[...].reshape(1, -1), wnei_ref[...],
                       precision=hp)
        agg = (jnp.dot(ax_ref[0], wxn[:dh, :], precision=hp)
               + jnp.dot(ax_ref[1], wxn[dh:, :], precision=hp)
               + deg_v.reshape(-1, 1) * bnei
               + jnp.dot(ae, wedge_ref[...], precision=hp))
        h2 = jnp.maximum(jnp.dot(h, wself_ref[...], precision=hp) + agg, 0.0)
        t = jnp.dot(h2, w1_ref[...], precision=hp) + b1_ref[...][None, :]
        t_ref[...] = t

        @pl.when(i == 0)
        def _():
            s_ref[...] = jnp.zeros_like(s_ref)

        s_ref[0, :] += jnp.sum(t, axis=0)
        s_ref[1, :] += jnp.sum(t * t, axis=0)

    t_full, sums = pl.pallas_call(
        body1,
        grid=(nb,),
        in_specs=[
            pl.BlockSpec((rb, d), lambda i: (i, 0)),
            pl.BlockSpec((2, rb, dh), lambda i: (0, i, 0)),
            pl.BlockSpec((2, rb, dep), lambda i: (0, i, 0)),
            pl.BlockSpec((d, d), lambda i: (0, 0)),
            pl.BlockSpec((d,), lambda i: (0,)),
            pl.BlockSpec((d, d), lambda i: (0, 0)),
            pl.BlockSpec((d, d), lambda i: (0, 0)),
            pl.BlockSpec((de, d), lambda i: (0, 0)),
            pl.BlockSpec((d, h_w), lambda i: (0, 0)),
            pl.BlockSpec((h_w,), lambda i: (0,)),
        ],
        out_specs=[
            pl.BlockSpec((rb, h_w), lambda i: (i, 0)),
            pl.BlockSpec((2, h_w), lambda i: (0, 0)),
        ],
        out_shape=[
            jax.ShapeDtypeStruct((n, h_w), jnp.float32),
            jax.ShapeDtypeStruct((2, h_w), jnp.float32),
        ],
    )(x, agx, age, W_enc, b_enc, W_self, W_nei, W_edge, W1, b1)

    def body2(t_ref, s_ref, g_ref, bt_ref, w2_ref, b2_ref, o_ref):
        t = t_ref[...]
        mean = s_ref[0, :][None, :] * (1.0 / n)
        var = s_ref[1, :][None, :] * (1.0 / n) - mean * mean
        tn = ((t - mean) * lax.rsqrt(var + 1e-5) * g_ref[...][None, :]
              + bt_ref[...][None, :])
        h3 = jnp.maximum(tn, 0.0)
        o_ref[...] = (jnp.dot(h3, w2_ref[...], precision=hp)
                      + b2_ref[...][None, :])

    return pl.pallas_call(
        body2,
        grid=(nb,),
        in_specs=[
            pl.BlockSpec((rb, h_w), lambda i: (i, 0)),
            pl.BlockSpec((2, h_w), lambda i: (0, 0)),
            pl.BlockSpec((h_w,), lambda i: (0,)),
            pl.BlockSpec((h_w,), lambda i: (0,)),
            pl.BlockSpec((h_w, out_w), lambda i: (0, 0)),
            pl.BlockSpec((out_w,), lambda i: (0,)),
        ],
        out_specs=pl.BlockSpec((rb, out_w), lambda i: (i, 0)),
        out_shape=jax.ShapeDtypeStruct((n, out_w), jnp.float32),
    )(t_full, sums, gamma, beta, W2, b2)


def kernel(x, edge_index, edge_attr, batch, W_enc, b_enc, W_self, W_nei,
           W_edge, W1, b1, gamma, beta, W2, b2):
    n, d = x.shape
    e, de = edge_attr.shape
    dh = d // 2
    blk = _NS * _K
    npad = -(-n // blk) * blk
    if npad == n:
        npad += blk
    chunks = -(-e // (_NS * _K))
    chunks = -(-chunks // _GRP) * _GRP
    epad = _NS * chunks * _K
    pad = epad - e

    src = edge_index[0].astype(jnp.int32)
    dst = edge_index[1].astype(jnp.int32)
    srcp = jnp.concatenate([src, jnp.zeros((pad,), jnp.int32)])
    dstp = jnp.concatenate([dst, jnp.full((pad,), npad - 1, jnp.int32)])
    src2 = jnp.concatenate([srcp, srcp + n]).reshape(_NC * _NS * chunks, _K)
    dst2 = jnp.concatenate([dstp, dstp]).reshape(_NC * _NS * chunks, _K)
    x2 = jnp.concatenate([x[:, :dh], x[:, dh:]], axis=0)

    dep = -(-de // 16) * 16
    ea0 = jnp.concatenate(
        [edge_attr.astype(jnp.float32),
         jnp.zeros((e, dep - de), jnp.float32)], axis=1)
    ea1 = jnp.concatenate(
        [jnp.ones((e, 1), jnp.float32),
         jnp.zeros((e, dep - 1), jnp.float32)], axis=1)
    zpad = jnp.zeros((pad, dep), jnp.float32)
    ea2 = jnp.concatenate([ea0, zpad, ea1, zpad])
    ea2 = ea2.reshape(_NC * _NS * chunks, _K, dep)

    agx, age = _sc_aggregate(x2, src2, dst2, ea2, npad)
    agx = agx[:, :n, :]
    age = age[:, :n, :]

    rb = n
    for cand in range(min(1024, n), 0, -1):
        if n % cand == 0:
            rb = cand
            break
    out = _dense(n, rb, de, x, agx, age, W_enc, b_enc, W_self, W_nei,
                 W_edge, W1, b1, gamma, beta, W2, b2)
    return out.reshape(-1, 8, 4)


# speedup vs baseline: 2.8726x; 1.1376x over previous
import functools

import jax
import jax.numpy as jnp
from jax import lax
from jax.experimental import pallas as pl
from jax.experimental.pallas import tpu as pltpu
from jax.experimental.pallas import tpu_sc as plsc

_NC = 2
_NS = 16
_K = 128
_GRP = 8


def _sc_aggregate(x2, src2, dst2, ea2, npad):
    dh = x2.shape[1]
    dep = ea2.shape[-1]
    chunks = src2.shape[0] // (_NC * _NS)
    rows_per_tile = npad // _NS
    nslab = rows_per_tile // _K
    mesh = plsc.VectorSubcoreMesh(core_axis_name="c", subcore_axis_name="s")

    nbuf = 4

    @functools.partial(
        pl.kernel,
        out_type=(
            jax.ShapeDtypeStruct((_NC, npad, dh), jnp.float32),
            jax.ShapeDtypeStruct((_NC, npad, dep), jnp.float32),
        ),
        mesh=mesh,
        compiler_params=pltpu.CompilerParams(use_tc_tiling_on_sc=False),
        scratch_types=(
            [pltpu.VMEM((_GRP, _K), jnp.int32)] * 2 +
            [pltpu.VMEM((_K, dh), jnp.float32)] * nbuf +
            [pltpu.VMEM((_K, dep), jnp.float32)] * 2 +
            [pltpu.VMEM_SHARED((npad, dh), jnp.float32),
             pltpu.VMEM_SHARED((npad, dep), jnp.float32)] +
            [pltpu.SemaphoreType.DMA] * (2 * nbuf + 2)
        ),
    )
    def sc(x_hbm, src_hbm, dst_hbm, ea_hbm,
           outx_hbm, oute_hbm, *scr):
        sidx_g, didx_g = scr[0], scr[1]
        rows = list(scr[2:2 + nbuf])
        eab = list(scr[2 + nbuf:4 + nbuf])
        accx_sh, acce_sh = scr[4 + nbuf], scr[5 + nbuf]
        gsem = list(scr[6 + nbuf:6 + 2 * nbuf])
        sxsem = list(scr[6 + 2 * nbuf:6 + 3 * nbuf])
        sesem = list(scr[6 + 3 * nbuf:8 + 3 * nbuf])
        cid = lax.axis_index("c")
        sid = lax.axis_index("s")
        row0 = (cid * _NS + sid) * chunks
        elast = ea_hbm.shape[0] - 1
        erow0 = jnp.where(cid == 0, sid * chunks, elast)
        estep = jnp.where(cid == 0, 1, 0)
        r0 = sid * rows_per_tile
        zero16 = jnp.zeros((16,), jnp.float32)
        rows0_v = rows[0]
        ea0_v = eab[0]

        def zrow(i, c):
            for c8 in range(dh // 16):
                rows0_v[i, pl.ds(c8 * 16, 16)] = zero16
            for c8 in range(dep // 16):
                ea0_v[i, pl.ds(c8 * 16, 16)] = zero16
            return c
        lax.fori_loop(0, _K, zrow, 0)

        def zslab(i, c):
            base = r0 + i * _K
            pltpu.sync_copy(rows0_v, accx_sh.at[pl.ds(base, _K)])
            pltpu.sync_copy(ea0_v, acce_sh.at[pl.ds(base, _K)])
            return c
        lax.fori_loop(0, nslab, zslab, 0)

        plsc.subcore_barrier()

        def group(g, carry):
            gb = row0 + g * _GRP
            eb = erow0 + g * _GRP * estep
            pltpu.sync_copy(src_hbm.at[pl.ds(gb, _GRP)], sidx_g)
            pltpu.sync_copy(dst_hbm.at[pl.ds(gb, _GRP)], didx_g)

            gds = {}
            pend_sx = {}
            pend_se = {}
            for jj in range(min(nbuf - 1, _GRP)):
                gds[jj] = pltpu.async_copy(
                    x_hbm.at[sidx_g.at[jj]], rows[jj], gsem[jj])
            for j in range(_GRP):
                b = j % nbuf
                jn = j + nbuf - 1
                if jn < _GRP:
                    bn = jn % nbuf
                    if j >= 1:
                        pend_sx[j - 1].wait()
                    gds[jn] = pltpu.async_copy(
                        x_hbm.at[sidx_g.at[jn]], rows[bn], gsem[bn])
                gds[j].wait()
                pend_sx[j] = pltpu.async_copy(
                    rows[b], accx_sh.at[didx_g.at[j]], sxsem[b], add=True)
                pb = j & 1
                if j >= 2:
                    pend_se[j - 2].wait()
                pltpu.sync_copy(ea_hbm.at[eb + j * estep], eab[pb])
                pend_se[j] = pltpu.async_copy(
                    eab[pb], acce_sh.at[didx_g.at[j]], sesem[pb], add=True)
            for j in range(max(0, _GRP - nbuf), _GRP):
                pend_sx[j].wait()
            pend_se[_GRP - 2].wait()
            pend_se[_GRP - 1].wait()
            return carry

        lax.fori_loop(0, chunks // _GRP, group, 0)
        plsc.subcore_barrier()

        def rslab(i, c):
            base = r0 + i * _K
            pltpu.sync_copy(accx_sh.at[pl.ds(base, _K)], rows0_v)
            pltpu.sync_copy(rows0_v, outx_hbm.at[cid, pl.ds(base, _K)])
            pltpu.sync_copy(acce_sh.at[pl.ds(base, _K)], ea0_v)
            pltpu.sync_copy(ea0_v, oute_hbm.at[cid, pl.ds(base, _K)])
            return c
        lax.fori_loop(0, nslab, rslab, 0)

    return sc(x2, src2, dst2, ea2)


def _dense(n, rb, de, x, agx, age, W_enc, b_enc, W_self, W_nei, W_edge,
           W1, b1, gamma, beta, W2, b2):
    d = x.shape[1]
    dh = agx.shape[2]
    dep = age.shape[2]
    h_w = W1.shape[1]
    out_w = W2.shape[1]
    nb = n // rb
    hp = lax.Precision.HIGHEST

    def body1(x_ref, ax_ref, ae_ref, wenc_ref, benc_ref, wself_ref,
              wnei_ref, wedge_ref, w1_ref, b1_ref, t_ref, s_ref):
        i = pl.program_id(0)
        h = (jnp.dot(x_ref[...], wenc_ref[...], precision=hp)
             + benc_ref[...][None, :])
        ae = ae_ref[0][:, :de]
        deg_v = ae_ref[1][:, 0]
        wxn = jnp.dot(wenc_ref[...], wnei_ref[...], precision=hp)
        bnei = jnp.dot(benc_ref[...].reshape(1, -1), wnei_ref[...],
                       precision=hp)
        agg = (jnp.dot(ax_ref[0], wxn[:dh, :], precision=hp)
               + jnp.dot(ax_ref[1], wxn[dh:, :], precision=hp)
               + deg_v.reshape(-1, 1) * bnei
               + jnp.dot(ae, wedge_ref[...], precision=hp))
        h2 = jnp.maximum(jnp.dot(h, wself_ref[...], precision=hp) + agg, 0.0)
        t = jnp.dot(h2, w1_ref[...], precision=hp) + b1_ref[...][None, :]
        t_ref[...] = t

        @pl.when(i == 0)
        def _():
            s_ref[...] = jnp.zeros_like(s_ref)

        s_ref[0, :] += jnp.sum(t, axis=0)
        s_ref[1, :] += jnp.sum(t * t, axis=0)

    t_full, sums = pl.pallas_call(
        body1,
        grid=(nb,),
        in_specs=[
            pl.BlockSpec((rb, d), lambda i: (i, 0)),
            pl.BlockSpec((2, rb, dh), lambda i: (0, i, 0)),
            pl.BlockSpec((2, rb, dep), lambda i: (0, i, 0)),
            pl.BlockSpec((d, d), lambda i: (0, 0)),
            pl.BlockSpec((d,), lambda i: (0,)),
            pl.BlockSpec((d, d), lambda i: (0, 0)),
            pl.BlockSpec((d, d), lambda i: (0, 0)),
            pl.BlockSpec((de, d), lambda i: (0, 0)),
            pl.BlockSpec((d, h_w), lambda i: (0, 0)),
            pl.BlockSpec((h_w,), lambda i: (0,)),
        ],
        out_specs=[
            pl.BlockSpec((rb, h_w), lambda i: (i, 0)),
            pl.BlockSpec((2, h_w), lambda i: (0, 0)),
        ],
        out_shape=[
            jax.ShapeDtypeStruct((n, h_w), jnp.float32),
            jax.ShapeDtypeStruct((2, h_w), jnp.float32),
        ],
    )(x, agx, age, W_enc, b_enc, W_self, W_nei, W_edge, W1, b1)

    def body2(t_ref, s_ref, g_ref, bt_ref, w2_ref, b2_ref, o_ref):
        t = t_ref[...]
        mean = s_ref[0, :][None, :] * (1.0 / n)
        var = s_ref[1, :][None, :] * (1.0 / n) - mean * mean
        tn = ((t - mean) * lax.rsqrt(var + 1e-5) * g_ref[...][None, :]
              + bt_ref[...][None, :])
        h3 = jnp.maximum(tn, 0.0)
        o_ref[...] = (jnp.dot(h3, w2_ref[...], precision=hp)
                      + b2_ref[...][None, :])

    return pl.pallas_call(
        body2,
        grid=(nb,),
        in_specs=[
            pl.BlockSpec((rb, h_w), lambda i: (i, 0)),
            pl.BlockSpec((2, h_w), lambda i: (0, 0)),
            pl.BlockSpec((h_w,), lambda i: (0,)),
            pl.BlockSpec((h_w,), lambda i: (0,)),
            pl.BlockSpec((h_w, out_w), lambda i: (0, 0)),
            pl.BlockSpec((out_w,), lambda i: (0,)),
        ],
        out_specs=pl.BlockSpec((rb, out_w), lambda i: (i, 0)),
        out_shape=jax.ShapeDtypeStruct((n, out_w), jnp.float32),
    )(t_full, sums, gamma, beta, W2, b2)


def kernel(x, edge_index, edge_attr, batch, W_enc, b_enc, W_self, W_nei,
           W_edge, W1, b1, gamma, beta, W2, b2):
    n, d = x.shape
    e, de = edge_attr.shape
    dh = d // 2
    blk = _NS * _K
    npad = -(-n // blk) * blk
    if npad == n:
        npad += blk
    chunks = -(-e // (_NS * _K))
    chunks = -(-chunks // _GRP) * _GRP
    epad = _NS * chunks * _K
    pad = epad - e

    src = edge_index[0].astype(jnp.int32)
    dst = edge_index[1].astype(jnp.int32)
    srcp = jnp.concatenate([src, jnp.zeros((pad,), jnp.int32)])
    dstp = jnp.concatenate([dst, jnp.full((pad,), npad - 1, jnp.int32)])
    src2 = jnp.concatenate([srcp, srcp + n]).reshape(_NC * _NS * chunks, _K)
    dst2 = jnp.concatenate([dstp, dstp]).reshape(_NC * _NS * chunks, _K)
    x2 = jnp.concatenate([x[:, :dh], x[:, dh:]], axis=0)

    dep = -(-de // 16) * 16
    ea0 = edge_attr.astype(jnp.float32)
    if dep > de:
        ea0 = jnp.concatenate([ea0, jnp.zeros((e, dep - de), jnp.float32)],
                              axis=1)
    ones_slab = jnp.zeros((_K, dep), jnp.float32).at[:, 0].set(1.0)
    ea2 = jnp.concatenate([ea0, jnp.zeros((pad, dep), jnp.float32),
                           ones_slab])
    ea2 = ea2.reshape(_NS * chunks + 1, _K, dep)

    agx, age = _sc_aggregate(x2, src2, dst2, ea2, npad)
    agx = agx[:, :n, :]
    age = age[:, :n, :]

    rb = n
    for cand in range(min(1024, n), 0, -1):
        if n % cand == 0:
            rb = cand
            break
    out = _dense(n, rb, de, x, agx, age, W_enc, b_enc, W_self, W_nei,
                 W_edge, W1, b1, gamma, beta, W2, b2)
    return out.reshape(-1, 8, 4)
